# Initial kernel scaffold; baseline (speedup 1.0000x reference)
import functools

import jax
import jax.numpy as jnp
from jax import lax
from jax.experimental import pallas as pl
from jax.experimental.pallas import tpu as pltpu
from jax.experimental.pallas import tpu_sc as plsc

N, E, D, DE = 10000, 320000, 128, 4
NC, NS = 2, 16
NT = NC * NS
C = 128
NCHUNK = E // C
ROWS_PER_TILE = N // NS
ZR = 125


def _gate_body(ea_ref, we_ref, beg_ref, o_ref):
    z = jnp.dot(ea_ref[...], we_ref[...], preferred_element_type=jnp.float32)
    o_ref[...] = jax.nn.sigmoid(z + beg_ref[0, 0])


def _edge_gate(edge_attr, we, beg):
    BE = 8000
    return pl.pallas_call(
        _gate_body,
        grid=(E // BE,),
        in_specs=[
            pl.BlockSpec((BE, DE), lambda i: (i, 0)),
            pl.BlockSpec((DE, 1), lambda i: (0, 0)),
            pl.BlockSpec((1, 1), lambda i: (0, 0)),
        ],
        out_specs=pl.BlockSpec((BE, 1), lambda i: (i, 0)),
        out_shape=jax.ShapeDtypeStruct((E, 1), jnp.float32),
    )(edge_attr, we, beg.reshape(1, 1))


def _sc_body(x_hbm, ei_hbm, gate_hbm, out_hbm, src_v, dst_v, gate_v, rows_v, shared, sem):
    cid = lax.axis_index("c")
    sid = lax.axis_index("s")
    wid = sid * NC + cid

    def _zrow(r, carry):
        for f in range(D // 16):
            rows_v[r, pl.ds(f * 16, 16)] = jnp.zeros((16,), jnp.float32)
        return carry

    lax.fori_loop(0, C, _zrow, 0)
    for j in range(ROWS_PER_TILE // ZR):
        pltpu.sync_copy(rows_v.at[pl.ds(0, ZR)],
                        shared.at[pl.ds(sid * ROWS_PER_TILE + j * ZR, ZR)])
    plsc.subcore_barrier()

    nch = (NCHUNK - 1 - wid) // NT + 1

    def _chunk(k, carry):
        base = (wid + k * NT) * C
        pltpu.sync_copy(ei_hbm.at[0, pl.ds(base, C)], src_v)
        pltpu.sync_copy(ei_hbm.at[1, pl.ds(base, C)], dst_v)
        pltpu.sync_copy(gate_hbm.at[pl.ds(base, C)], gate_v)
        pltpu.async_copy(x_hbm.at[src_v], rows_v, sem).wait()

        def _scale(e, c2):
            g = plsc.load_gather(gate_v, [jnp.full((16,), e, jnp.int32)])
            for f in range(D // 16):
                sl = pl.ds(f * 16, 16)
                rows_v[e, sl] = rows_v[e, sl] * g
            return c2

        lax.fori_loop(0, C, _scale, 0)
        pltpu.sync_copy(rows_v, shared.at[dst_v], add=True)
        return carry

    lax.fori_loop(0, nch, _chunk, 0)
    plsc.subcore_barrier()

    for j in range(ROWS_PER_TILE // ZR):
        r0 = sid * ROWS_PER_TILE + j * ZR
        pltpu.sync_copy(shared.at[pl.ds(r0, ZR)], out_hbm.at[cid, pl.ds(r0, ZR)])


def _sc_aggregate(x, edge_index, gate):
    mesh = plsc.VectorSubcoreMesh(core_axis_name="c", subcore_axis_name="s")
    return pl.kernel(
        _sc_body,
        out_type=jax.ShapeDtypeStruct((NC, N, D), jnp.float32),
        mesh=mesh,
        scratch_types=[
            pltpu.VMEM((C,), jnp.int32),
            pltpu.VMEM((C,), jnp.int32),
            pltpu.VMEM((C,), jnp.float32),
            pltpu.VMEM((C, D), jnp.float32),
            pltpu.VMEM_SHARED((N, D), jnp.float32),
            pltpu.SemaphoreType.DMA,
        ],
    )(x, edge_index, gate)


def _out_body(x_ref, p0_ref, p1_ref, wn_ref, bn_ref, o_ref):
    w = wn_ref[...]
    agg = p0_ref[...] + p1_ref[...]
    o_ref[...] = (jnp.dot(x_ref[...], w[:D], preferred_element_type=jnp.float32)
                  + jnp.dot(agg, w[D:], preferred_element_type=jnp.float32)
                  + bn_ref[...])


def _node_update(x, p0, p1, Wn, bn):
    BN = 2000
    return pl.pallas_call(
        _out_body,
        grid=(N // BN,),
        in_specs=[
            pl.BlockSpec((BN, D), lambda i: (i, 0)),
            pl.BlockSpec((BN, D), lambda i: (i, 0)),
            pl.BlockSpec((BN, D), lambda i: (i, 0)),
            pl.BlockSpec((2 * D, D), lambda i: (0, 0)),
            pl.BlockSpec((1, D), lambda i: (0, 0)),
        ],
        out_specs=pl.BlockSpec((BN, D), lambda i: (i, 0)),
        out_shape=jax.ShapeDtypeStruct((N, D), jnp.float32),
    )(x, p0, p1, Wn, bn.reshape(1, D))


def kernel(x, edge_index, edge_attr, we, beg, Wn, bn):
    gate = _edge_gate(edge_attr, we, beg).reshape(E)
    partials = _sc_aggregate(x, edge_index, gate)
    return _node_update(x, partials[0], partials[1], Wn, bn)


# speedup vs baseline: 3.1355x; 3.1355x over previous
import jax
import jax.numpy as jnp
from jax import lax
from jax.experimental import pallas as pl
from jax.experimental.pallas import tpu as pltpu
from jax.experimental.pallas import tpu_sc as plsc

N, E, D, DE = 10000, 320000, 128, 4
NC, NS = 2, 16
NT = NC * NS
C = 128
NCHUNK = E // C
NPAD = 10240
ROWS_PER_TILE = NPAD // NS


def _gate_body(ea_ref, we_ref, beg_ref, o_ref):
    z = jnp.dot(ea_ref[...], we_ref[...], preferred_element_type=jnp.float32)
    o_ref[...] = jax.nn.sigmoid(z + beg_ref[0, 0])


def _edge_gate(edge_attr, we, beg):
    BE = 8000
    return pl.pallas_call(
        _gate_body,
        grid=(E // BE,),
        in_specs=[
            pl.BlockSpec((BE, DE), lambda i: (i, 0)),
            pl.BlockSpec((DE, 1), lambda i: (0, 0)),
            pl.BlockSpec((1, 1), lambda i: (0, 0)),
        ],
        out_specs=pl.BlockSpec((BE, 1), lambda i: (i, 0)),
        out_shape=jax.ShapeDtypeStruct((E, 1), jnp.float32),
    )(edge_attr, we, beg.reshape(1, 1))


def _sc_body(x_hbm, src_hbm, dst_hbm, gate_hbm, out_hbm,
             src_v, dst_v, gate_v, rows_v, shared, sem):
    cid = lax.axis_index("c")
    sid = lax.axis_index("s")
    wid = sid * NC + cid

    def _zrow(r, carry):
        for f in range(D // 16):
            rows_v[r, pl.ds(f * 16, 16)] = jnp.zeros((16,), jnp.float32)
        return carry

    lax.fori_loop(0, C, _zrow, 0)
    for j in range(ROWS_PER_TILE // C):
        pltpu.sync_copy(rows_v, shared.at[pl.ds(sid * ROWS_PER_TILE + j * C, C)])
    plsc.subcore_barrier()

    nch = (NCHUNK - 1 - wid) // NT + 1

    def _chunk(k, carry):
        base = (wid + k * NT) * C
        pltpu.sync_copy(src_hbm.at[pl.ds(base, C)], src_v)
        pltpu.sync_copy(dst_hbm.at[pl.ds(base, C)], dst_v)
        pltpu.sync_copy(gate_hbm.at[pl.ds(base, C)], gate_v)
        pltpu.async_copy(x_hbm.at[src_v], rows_v, sem).wait()

        def _scale(j, c2):
            g16 = gate_v[pl.ds(j * 16, 16)]
            for t in range(16):
                e = j * 16 + t
                g = jnp.full((16,), g16[t], jnp.float32)
                for f in range(D // 16):
                    sl = pl.ds(f * 16, 16)
                    rows_v[e, sl] = rows_v[e, sl] * g
            return c2

        lax.fori_loop(0, C // 16, _scale, 0)
        pltpu.sync_copy(rows_v, shared.at[dst_v], add=True)
        return carry

    lax.fori_loop(0, nch, _chunk, 0)
    plsc.subcore_barrier()

    for j in range(ROWS_PER_TILE // C):
        r0 = sid * ROWS_PER_TILE + j * C
        pltpu.sync_copy(shared.at[pl.ds(r0, C)], out_hbm.at[cid, pl.ds(r0, C)])


def _sc_aggregate(x, src, dst, gate):
    mesh = plsc.VectorSubcoreMesh(core_axis_name="c", subcore_axis_name="s")
    return pl.kernel(
        _sc_body,
        out_type=jax.ShapeDtypeStruct((NC, NPAD, D), jnp.float32),
        mesh=mesh,
        scratch_types=[
            pltpu.VMEM((C,), jnp.int32),
            pltpu.VMEM((C,), jnp.int32),
            pltpu.VMEM((C,), jnp.float32),
            pltpu.VMEM((C, D), jnp.float32),
            pltpu.VMEM_SHARED((NPAD, D), jnp.float32),
            pltpu.SemaphoreType.DMA,
        ],
    )(x, src, dst, gate)


def _out_body(x_ref, p0_ref, p1_ref, wn_ref, bn_ref, o_ref):
    w = wn_ref[...]
    agg = p0_ref[0] + p1_ref[0]
    o_ref[...] = (jnp.dot(x_ref[...], w[:D], preferred_element_type=jnp.float32)
                  + jnp.dot(agg, w[D:], preferred_element_type=jnp.float32)
                  + bn_ref[...])


def _node_update(x, partials, Wn, bn):
    BN = 2000
    return pl.pallas_call(
        _out_body,
        grid=(N // BN,),
        in_specs=[
            pl.BlockSpec((BN, D), lambda i: (i, 0)),
            pl.BlockSpec((1, BN, D), lambda i: (0, i, 0)),
            pl.BlockSpec((1, BN, D), lambda i: (1, i, 0)),
            pl.BlockSpec((2 * D, D), lambda i: (0, 0)),
            pl.BlockSpec((1, D), lambda i: (0, 0)),
        ],
        out_specs=pl.BlockSpec((BN, D), lambda i: (i, 0)),
        out_shape=jax.ShapeDtypeStruct((N, D), jnp.float32),
    )(x, partials, partials, Wn, bn.reshape(1, D))


def kernel(x, edge_index, edge_attr, we, beg, Wn, bn):
    src = edge_index[0]
    dst = edge_index[1]
    gate = _edge_gate(edge_attr, we, beg).reshape(E)
    partials = _sc_aggregate(x, src, dst, gate)
    return _node_update(x, partials, Wn, bn)


# speedup vs baseline: 3.1519x; 1.0052x over previous
import jax
import jax.numpy as jnp
from jax import lax
from jax.experimental import pallas as pl
from jax.experimental.pallas import tpu as pltpu
from jax.experimental.pallas import tpu_sc as plsc

N, E, D, DE = 10000, 320000, 128, 4
NC, NS = 2, 16
NT = NC * NS
C = 128
NCHUNK = E // C
NPAD = 10240
ROWS_PER_TILE = NPAD // NS


def _gate_body(ea_ref, we_ref, beg_ref, o_ref):
    z = jnp.dot(ea_ref[...], we_ref[...], preferred_element_type=jnp.float32)
    o_ref[...] = jax.nn.sigmoid(z + beg_ref[0, 0])


def _edge_gate(edge_attr, we, beg):
    BE = 8000
    return pl.pallas_call(
        _gate_body,
        grid=(E // BE,),
        in_specs=[
            pl.BlockSpec((BE, DE), lambda i: (i, 0)),
            pl.BlockSpec((DE, 1), lambda i: (0, 0)),
            pl.BlockSpec((1, 1), lambda i: (0, 0)),
        ],
        out_specs=pl.BlockSpec((BE, 1), lambda i: (i, 0)),
        out_shape=jax.ShapeDtypeStruct((E, 1), jnp.float32),
    )(edge_attr, we, beg.reshape(1, 1))


def _sc_body(x_hbm, src_hbm, dst_hbm, gate_hbm, out0_hbm, out1_hbm,
             src_v, dst_v, gate_v, rows_v, shared, sem):
    cid = lax.axis_index("c")
    sid = lax.axis_index("s")
    wid = sid * NC + cid

    def _zrow(r, carry):
        for f in range(D // 16):
            rows_v[r, pl.ds(f * 16, 16)] = jnp.zeros((16,), jnp.float32)
        return carry

    lax.fori_loop(0, C, _zrow, 0)
    for j in range(ROWS_PER_TILE // C):
        pltpu.sync_copy(rows_v, shared.at[pl.ds(sid * ROWS_PER_TILE + j * C, C)])
    plsc.subcore_barrier()

    nch = (NCHUNK - 1 - wid) // NT + 1

    def _chunk(k, carry):
        base = (wid + k * NT) * C
        pltpu.sync_copy(src_hbm.at[pl.ds(base, C)], src_v)
        pltpu.sync_copy(dst_hbm.at[pl.ds(base, C)], dst_v)
        pltpu.sync_copy(gate_hbm.at[pl.ds(base, C)], gate_v)
        pltpu.async_copy(x_hbm.at[src_v], rows_v, sem).wait()

        def _scale(j, c2):
            g16 = gate_v[pl.ds(j * 16, 16)]
            for t in range(16):
                e = j * 16 + t
                g = jnp.full((16,), g16[t], jnp.float32)
                for f in range(D // 16):
                    sl = pl.ds(f * 16, 16)
                    rows_v[e, sl] = rows_v[e, sl] * g
            return c2

        lax.fori_loop(0, C // 16, _scale, 0)
        pltpu.sync_copy(rows_v, shared.at[dst_v], add=True)
        return carry

    lax.fori_loop(0, nch, _chunk, 0)
    plsc.subcore_barrier()

    @pl.when(cid == 0)
    def _():
        for j in range(ROWS_PER_TILE // C):
            r0 = sid * ROWS_PER_TILE + j * C
            pltpu.sync_copy(shared.at[pl.ds(r0, C)], out0_hbm.at[pl.ds(r0, C)])

    @pl.when(cid == 1)
    def _():
        for j in range(ROWS_PER_TILE // C):
            r0 = sid * ROWS_PER_TILE + j * C
            pltpu.sync_copy(shared.at[pl.ds(r0, C)], out1_hbm.at[pl.ds(r0, C)])


def _sc_aggregate(x, src, dst, gate):
    mesh = plsc.VectorSubcoreMesh(core_axis_name="c", subcore_axis_name="s")
    return pl.kernel(
        _sc_body,
        out_type=(jax.ShapeDtypeStruct((NPAD, D), jnp.float32),
                  jax.ShapeDtypeStruct((NPAD, D), jnp.float32)),
        mesh=mesh,
        scratch_types=[
            pltpu.VMEM((C,), jnp.int32),
            pltpu.VMEM((C,), jnp.int32),
            pltpu.VMEM((C,), jnp.float32),
            pltpu.VMEM((C, D), jnp.float32),
            pltpu.VMEM_SHARED((NPAD, D), jnp.float32),
            pltpu.SemaphoreType.DMA,
        ],
    )(x, src, dst, gate)


def _out_body(x_ref, p0_ref, p1_ref, wn_ref, bn_ref, o_ref):
    w = wn_ref[...]
    agg = p0_ref[...] + p1_ref[...]
    o_ref[...] = (jnp.dot(x_ref[...], w[:D], preferred_element_type=jnp.float32)
                  + jnp.dot(agg, w[D:], preferred_element_type=jnp.float32)
                  + bn_ref[...])


def _node_update(x, p0, p1, Wn, bn):
    BN = 2000
    return pl.pallas_call(
        _out_body,
        grid=(N // BN,),
        in_specs=[
            pl.BlockSpec((BN, D), lambda i: (i, 0)),
            pl.BlockSpec((BN, D), lambda i: (i, 0)),
            pl.BlockSpec((BN, D), lambda i: (i, 0)),
            pl.BlockSpec((2 * D, D), lambda i: (0, 0)),
            pl.BlockSpec((1, D), lambda i: (0, 0)),
        ],
        out_specs=pl.BlockSpec((BN, D), lambda i: (i, 0)),
        out_shape=jax.ShapeDtypeStruct((N, D), jnp.float32),
    )(x, p0, p1, Wn, bn.reshape(1, D))


def kernel(x, edge_index, edge_attr, we, beg, Wn, bn):
    src = edge_index[0]
    dst = edge_index[1]
    gate = _edge_gate(edge_attr, we, beg).reshape(E)
    p0, p1 = _sc_aggregate(x, src, dst, gate)
    return _node_update(x, p0, p1, Wn, bn)


# speedup vs baseline: 3.8767x; 1.2300x over previous
import jax
import jax.numpy as jnp
from jax import lax
from jax.experimental import pallas as pl
from jax.experimental.pallas import tpu as pltpu
from jax.experimental.pallas import tpu_sc as plsc

N, E, D, DE = 10000, 320000, 128, 4
NC, NS = 2, 16
NT = NC * NS
C = 128
NCHUNK = E // C
NPAD = 10240
ROWS_PER_TILE = NPAD // NS


def _sc_body(x_hbm, ei_hbm, ea_hbm, wb_hbm, out0_hbm, out1_hbm,
             src_v, dst_v, ea_v, wb_v, rows_v, shared, sem):
    cid = lax.axis_index("c")
    sid = lax.axis_index("s")
    wid = sid * NC + cid

    pltpu.sync_copy(wb_hbm, wb_v)
    wb16 = wb_v[...]
    w0 = jnp.full((16,), wb16[0], jnp.float32)
    w1 = jnp.full((16,), wb16[1], jnp.float32)
    w2 = jnp.full((16,), wb16[2], jnp.float32)
    w3 = jnp.full((16,), wb16[3], jnp.float32)
    bg = jnp.full((16,), wb16[4], jnp.float32)
    one = jnp.ones((16,), jnp.float32)

    def _zrow(r, carry):
        for f in range(D // 16):
            rows_v[r, pl.ds(f * 16, 16)] = jnp.zeros((16,), jnp.float32)
        return carry

    lax.fori_loop(0, C, _zrow, 0)
    for j in range(ROWS_PER_TILE // C):
        pltpu.sync_copy(rows_v, shared.at[pl.ds(sid * ROWS_PER_TILE + j * C, C)])
    plsc.subcore_barrier()

    nch = (NCHUNK - 1 - wid) // NT + 1

    def _chunk(k, carry):
        base = (wid + k * NT) * C
        pltpu.sync_copy(ei_hbm.at[pl.ds(base, C)], src_v)
        pltpu.sync_copy(ei_hbm.at[pl.ds(E + base, C)], dst_v)
        for a in range(DE):
            pltpu.sync_copy(ea_hbm.at[pl.ds(a * E + base, C)],
                            ea_v.at[pl.ds(a * C, C)])
        pltpu.async_copy(x_hbm.at[src_v], rows_v, sem).wait()

        def _scale(j, c2):
            s16 = pl.ds(j * 16, 16)
            z = (w0 * ea_v[s16]
                 + w1 * ea_v[pl.ds(C + j * 16, 16)]
                 + w2 * ea_v[pl.ds(2 * C + j * 16, 16)]
                 + w3 * ea_v[pl.ds(3 * C + j * 16, 16)]
                 + bg)
            g16 = one / (one + jnp.exp(-z))
            for t in range(16):
                e = j * 16 + t
                g = jnp.full((16,), g16[t], jnp.float32)
                for f in range(D // 16):
                    sl = pl.ds(f * 16, 16)
                    rows_v[e, sl] = rows_v[e, sl] * g
            return c2

        lax.fori_loop(0, C // 16, _scale, 0)
        pltpu.sync_copy(rows_v, shared.at[dst_v], add=True)
        return carry

    lax.fori_loop(0, nch, _chunk, 0)
    plsc.subcore_barrier()

    @pl.when(cid == 0)
    def _():
        for j in range(ROWS_PER_TILE // C):
            r0 = sid * ROWS_PER_TILE + j * C
            pltpu.sync_copy(shared.at[pl.ds(r0, C)], out0_hbm.at[pl.ds(r0, C)])

    @pl.when(cid == 1)
    def _():
        for j in range(ROWS_PER_TILE // C):
            r0 = sid * ROWS_PER_TILE + j * C
            pltpu.sync_copy(shared.at[pl.ds(r0, C)], out1_hbm.at[pl.ds(r0, C)])


def _sc_aggregate(x, ei_flat, ea_t_flat, wb):
    mesh = plsc.VectorSubcoreMesh(core_axis_name="c", subcore_axis_name="s")
    return pl.kernel(
        _sc_body,
        out_type=(jax.ShapeDtypeStruct((NPAD, D), jnp.float32),
                  jax.ShapeDtypeStruct((NPAD, D), jnp.float32)),
        mesh=mesh,
        scratch_types=[
            pltpu.VMEM((C,), jnp.int32),
            pltpu.VMEM((C,), jnp.int32),
            pltpu.VMEM((DE * C,), jnp.float32),
            pltpu.VMEM((16,), jnp.float32),
            pltpu.VMEM((C, D), jnp.float32),
            pltpu.VMEM_SHARED((NPAD, D), jnp.float32),
            pltpu.SemaphoreType.DMA,
        ],
    )(x, ei_flat, ea_t_flat, wb)


def _out_body(x_ref, p0_ref, p1_ref, wn_ref, bn_ref, o_ref):
    w = wn_ref[...]
    agg = p0_ref[...] + p1_ref[...]
    o_ref[...] = (jnp.dot(x_ref[...], w[:D], preferred_element_type=jnp.float32)
                  + jnp.dot(agg, w[D:], preferred_element_type=jnp.float32)
                  + bn_ref[...])


def _node_update(x, p0, p1, Wn, bn):
    BN = 2000
    return pl.pallas_call(
        _out_body,
        grid=(N // BN,),
        in_specs=[
            pl.BlockSpec((BN, D), lambda i: (i, 0)),
            pl.BlockSpec((BN, D), lambda i: (i, 0)),
            pl.BlockSpec((BN, D), lambda i: (i, 0)),
            pl.BlockSpec((2 * D, D), lambda i: (0, 0)),
            pl.BlockSpec((1, D), lambda i: (0, 0)),
        ],
        out_specs=pl.BlockSpec((BN, D), lambda i: (i, 0)),
        out_shape=jax.ShapeDtypeStruct((N, D), jnp.float32),
    )(x, p0, p1, Wn, bn.reshape(1, D))


def kernel(x, edge_index, edge_attr, we, beg, Wn, bn):
    ei_flat = edge_index.reshape(2 * E)
    ea_t_flat = edge_attr.T.reshape(DE * E)
    wb = jnp.concatenate([we.reshape(DE), beg,
                          jnp.zeros((16 - DE - 1,), jnp.float32)])
    p0, p1 = _sc_aggregate(x, ei_flat, ea_t_flat, wb)
    return _node_update(x, p0, p1, Wn, bn)


# speedup vs baseline: 9.1781x; 2.3675x over previous
import jax
import jax.numpy as jnp
from jax import lax
from jax.experimental import pallas as pl
from jax.experimental.pallas import tpu as pltpu
from jax.experimental.pallas import tpu_sc as plsc

N, E, D, DE = 10000, 320000, 128, 4
NC, NS = 2, 16
NT = NC * NS
S = 1
C = 128 * S
NCHUNK = E // C
NPAD = 10240
ROWS_PER_TILE = NPAD // NS
PAIRS = (NCHUNK // NT + 2) // 2


def _sc_body(x_hbm, ei_hbm, ea_hbm, wb_hbm, out0_hbm, out1_hbm,
             src_a, dst_a, ea_a, rows_a, src_b, dst_b, ea_b, rows_b,
             wb_v, shared, isem_a, isem_b, gsem_a, gsem_b):
    cid = lax.axis_index("c")
    sid = lax.axis_index("s")
    wid = sid * NC + cid

    pltpu.sync_copy(wb_hbm, wb_v)
    wb16 = wb_v[...]
    ws = [jnp.full((16,), wb16[a], jnp.float32) for a in range(DE)]
    bg = jnp.full((16,), wb16[DE], jnp.float32)
    one = jnp.ones((16,), jnp.float32)

    def _zrow(r, carry):
        for f in range(D // 16):
            rows_a[r, pl.ds(f * 16, 16)] = jnp.zeros((16,), jnp.float32)
        return carry

    lax.fori_loop(0, 128, _zrow, 0)
    for j in range(ROWS_PER_TILE // 128):
        pltpu.sync_copy(rows_a.at[pl.ds(0, 128)],
                        shared.at[pl.ds(sid * ROWS_PER_TILE + j * 128, 128)])
    plsc.subcore_barrier()

    nch = (NCHUNK - 1 - wid) // NT + 1

    def chunk_base(k):
        return (wid + k * NT) * C

    def fetch_idx(k, src_v, dst_v, ea_v, sem, sync):
        base = chunk_base(k)
        copy = pltpu.sync_copy if sync else (
            lambda s_, d_: pltpu.async_copy(s_, d_, sem))
        for j in range(S):
            copy(ei_hbm.at[pl.ds(base + j * 128, 128)], src_v.at[j])
            copy(ei_hbm.at[pl.ds(E + base + j * 128, 128)], dst_v.at[j])
        copy(ea_hbm.at[:, pl.ds(base, C)], ea_v)

    def wait_idx(k, src_v, dst_v, ea_v, sem):
        base = chunk_base(k)
        for j in range(S):
            pltpu.make_async_copy(ei_hbm.at[pl.ds(base + j * 128, 128)],
                                  src_v.at[j], sem).wait()
            pltpu.make_async_copy(ei_hbm.at[pl.ds(E + base + j * 128, 128)],
                                  dst_v.at[j], sem).wait()
        pltpu.make_async_copy(ea_hbm.at[:, pl.ds(base, C)], ea_v, sem).wait()

    def start_gather(src_v, rows_v, sem):
        for j in range(S):
            pltpu.async_copy(x_hbm.at[src_v.at[j]],
                             rows_v.at[pl.ds(j * 128, 128)], sem)

    def wait_gather(src_v, rows_v, sem):
        for j in range(S):
            pltpu.make_async_copy(x_hbm.at[src_v.at[j]],
                                  rows_v.at[pl.ds(j * 128, 128)], sem).wait()

    def scale_rows(ea_v, rows_v):
        def _scale(j, c2):
            z = bg
            for a in range(DE):
                z = z + ws[a] * ea_v[a, pl.ds(j * 16, 16)]
            g16 = one / (one + jnp.exp(-z))
            for t in range(16):
                e = j * 16 + t
                g = jnp.full((16,), g16[t], jnp.float32)
                for f in range(D // 16):
                    sl = pl.ds(f * 16, 16)
                    rows_v[e, sl] = rows_v[e, sl] * g
            return c2

        lax.fori_loop(0, C // 16, _scale, 0)

    def scatter(dst_v, rows_v):
        for j in range(S):
            pltpu.sync_copy(rows_v.at[pl.ds(j * 128, 128)],
                            shared.at[dst_v.at[j]], add=True)

    fetch_idx(0, src_a, dst_a, ea_a, isem_a, sync=True)
    start_gather(src_a, rows_a, gsem_a)

    @pl.when(1 < nch)
    def _():
        fetch_idx(1, src_b, dst_b, ea_b, isem_b, sync=False)

    def sub_body(k, cur, nxt):
        (src_c, dst_c, ea_c, rows_c, gsem_c, _isem_c) = cur
        (src_n, dst_n, ea_n, rows_n, gsem_n, isem_n) = nxt

        @pl.when(k + 1 < nch)
        def _():
            wait_idx(k + 1, src_n, dst_n, ea_n, isem_n)
            start_gather(src_n, rows_n, gsem_n)

        wait_gather(src_c, rows_c, gsem_c)
        scale_rows(ea_c, rows_c)
        scatter(dst_c, rows_c)

        @pl.when(k + 2 < nch)
        def _():
            fetch_idx(k + 2, src_c, dst_c, ea_c, _isem_c, sync=False)

    buf_a = (src_a, dst_a, ea_a, rows_a, gsem_a, isem_a)
    buf_b = (src_b, dst_b, ea_b, rows_b, gsem_b, isem_b)

    def _pair(m, carry):
        k = m * 2

        @pl.when(k < nch)
        def _():
            sub_body(k, buf_a, buf_b)

        @pl.when(k + 1 < nch)
        def _():
            sub_body(k + 1, buf_b, buf_a)

        return carry

    lax.fori_loop(0, PAIRS, _pair, 0)
    plsc.subcore_barrier()

    @pl.when(cid == 0)
    def _():
        for j in range(ROWS_PER_TILE // 128):
            r0 = sid * ROWS_PER_TILE + j * 128
            pltpu.sync_copy(shared.at[pl.ds(r0, 128)], out0_hbm.at[pl.ds(r0, 128)])

    @pl.when(cid == 1)
    def _():
        for j in range(ROWS_PER_TILE // 128):
            r0 = sid * ROWS_PER_TILE + j * 128
            pltpu.sync_copy(shared.at[pl.ds(r0, 128)], out1_hbm.at[pl.ds(r0, 128)])


def _sc_aggregate(x, ei_flat, ea_t, wb):
    mesh = plsc.VectorSubcoreMesh(core_axis_name="c", subcore_axis_name="s")
    return pl.kernel(
        _sc_body,
        out_type=(jax.ShapeDtypeStruct((NPAD, D), jnp.float32),
                  jax.ShapeDtypeStruct((NPAD, D), jnp.float32)),
        mesh=mesh,
        scratch_types=[
            pltpu.VMEM((S, 128), jnp.int32),
            pltpu.VMEM((S, 128), jnp.int32),
            pltpu.VMEM((DE, C), jnp.float32),
            pltpu.VMEM((C, D), jnp.float32),
            pltpu.VMEM((S, 128), jnp.int32),
            pltpu.VMEM((S, 128), jnp.int32),
            pltpu.VMEM((DE, C), jnp.float32),
            pltpu.VMEM((C, D), jnp.float32),
            pltpu.VMEM((16,), jnp.float32),
            pltpu.VMEM_SHARED((NPAD, D), jnp.float32),
            pltpu.SemaphoreType.DMA,
            pltpu.SemaphoreType.DMA,
            pltpu.SemaphoreType.DMA,
            pltpu.SemaphoreType.DMA,
        ],
    )(x, ei_flat, ea_t, wb)


def _out_body(x_ref, p0_ref, p1_ref, wn_ref, bn_ref, o_ref):
    w = wn_ref[...]
    agg = p0_ref[...] + p1_ref[...]
    o_ref[...] = (jnp.dot(x_ref[...], w[:D], preferred_element_type=jnp.float32)
                  + jnp.dot(agg, w[D:], preferred_element_type=jnp.float32)
                  + bn_ref[...])


def _node_update(x, p0, p1, Wn, bn):
    BN = 2000
    return pl.pallas_call(
        _out_body,
        grid=(N // BN,),
        in_specs=[
            pl.BlockSpec((BN, D), lambda i: (i, 0)),
            pl.BlockSpec((BN, D), lambda i: (i, 0)),
            pl.BlockSpec((BN, D), lambda i: (i, 0)),
            pl.BlockSpec((2 * D, D), lambda i: (0, 0)),
            pl.BlockSpec((1, D), lambda i: (0, 0)),
        ],
        out_specs=pl.BlockSpec((BN, D), lambda i: (i, 0)),
        out_shape=jax.ShapeDtypeStruct((N, D), jnp.float32),
    )(x, p0, p1, Wn, bn.reshape(1, D))


def kernel(x, edge_index, edge_attr, we, beg, Wn, bn):
    ei_flat = edge_index.reshape(2 * E)
    ea_t = edge_attr.T
    wb = jnp.concatenate([we.reshape(DE), beg,
                          jnp.zeros((16 - DE - 1,), jnp.float32)])
    p0, p1 = _sc_aggregate(x, ei_flat, ea_t, wb)
    return _node_update(x, p0, p1, Wn, bn)


# speedup vs baseline: 11.2943x; 1.2306x over previous
import jax
import jax.numpy as jnp
from jax import lax
from jax.experimental import pallas as pl
from jax.experimental.pallas import tpu as pltpu
from jax.experimental.pallas import tpu_sc as plsc

N, E, D, DE = 10000, 320000, 128, 4
NC, NS = 2, 16
NT = NC * NS
C = 128
NCHUNK = E // C
NPAD = 10240
ROWS_PER_TILE = NPAD // NS
PAIRS = (NCHUNK // NT + 2) // 2
DSTN = 4


def _sc_body(x_hbm, ei_hbm, ea_hbm, wb_hbm, out0_hbm, out1_hbm,
             src_a, ea_a, rows_a, src_b, ea_b, rows_b, dst_v,
             wb_v, shared, isem_a, isem_b, gsem_a, gsem_b, ssem):
    cid = lax.axis_index("c")
    sid = lax.axis_index("s")
    wid = sid * NC + cid

    pltpu.sync_copy(wb_hbm, wb_v)
    wb16 = wb_v[...]
    ws = [jnp.full((16,), wb16[a], jnp.float32) for a in range(DE)]
    bg = jnp.full((16,), wb16[DE], jnp.float32)
    one = jnp.ones((16,), jnp.float32)

    def _zrow(r, carry):
        for f in range(D // 16):
            rows_a[r, pl.ds(f * 16, 16)] = jnp.zeros((16,), jnp.float32)
        return carry

    lax.fori_loop(0, C, _zrow, 0)
    for j in range(ROWS_PER_TILE // C):
        pltpu.sync_copy(rows_a, shared.at[pl.ds(sid * ROWS_PER_TILE + j * C, C)])
    plsc.subcore_barrier()

    nch = (NCHUNK - 1 - wid) // NT + 1

    def chunk_base(k):
        return (wid + k * NT) * C

    def fetch_idx(k, src_v, ea_v, sem, sync):
        base = chunk_base(k)
        copy = pltpu.sync_copy if sync else (
            lambda s_, d_: pltpu.async_copy(s_, d_, sem))
        copy(ei_hbm.at[pl.ds(base, C)], src_v.at[0])
        copy(ei_hbm.at[pl.ds(E + base, C)], dst_v.at[lax.rem(k, DSTN)])
        copy(ea_hbm.at[:, pl.ds(base, C)], ea_v)

    def wait_idx(k, src_v, ea_v, sem):
        base = chunk_base(k)
        pltpu.make_async_copy(ei_hbm.at[pl.ds(base, C)], src_v.at[0], sem).wait()
        pltpu.make_async_copy(ei_hbm.at[pl.ds(E + base, C)],
                              dst_v.at[lax.rem(k, DSTN)], sem).wait()
        pltpu.make_async_copy(ea_hbm.at[:, pl.ds(base, C)], ea_v, sem).wait()

    def scale_rows(ea_v, rows_v):
        def _scale(j, c2):
            z = bg
            for a in range(DE):
                z = z + ws[a] * ea_v[a, pl.ds(j * 16, 16)]
            g16 = one / (one + jnp.exp(-z))
            for t in range(16):
                e = j * 16 + t
                g = jnp.full((16,), g16[t], jnp.float32)
                for f in range(D // 16):
                    sl = pl.ds(f * 16, 16)
                    rows_v[e, sl] = rows_v[e, sl] * g
            return c2

        lax.fori_loop(0, C // 16, _scale, 0)

    def wait_scatter(rows_v):
        pltpu.make_async_copy(rows_v, shared.at[dst_v.at[0]], ssem).wait()

    fetch_idx(0, src_a, ea_a, isem_a, sync=True)
    pltpu.async_copy(x_hbm.at[src_a.at[0]], rows_a, gsem_a)

    @pl.when(1 < nch)
    def _():
        fetch_idx(1, src_b, ea_b, isem_b, sync=False)

    def sub_body(k, cur, nxt):
        (src_c, ea_c, rows_c, gsem_c, _isem_c) = cur
        (src_n, ea_n, rows_n, gsem_n, isem_n) = nxt

        @pl.when(k + 1 < nch)
        def _():
            wait_idx(k + 1, src_n, ea_n, isem_n)

            @pl.when(k >= 1)
            def _():
                wait_scatter(rows_n)

            pltpu.async_copy(x_hbm.at[src_n.at[0]], rows_n, gsem_n)

        pltpu.make_async_copy(x_hbm.at[src_c.at[0]], rows_c, gsem_c).wait()
        scale_rows(ea_c, rows_c)
        pltpu.async_copy(rows_c, shared.at[dst_v.at[lax.rem(k, DSTN)]], ssem,
                         add=True)

        @pl.when(k + 2 < nch)
        def _():
            fetch_idx(k + 2, src_c, ea_c, _isem_c, sync=False)

    buf_a = (src_a, ea_a, rows_a, gsem_a, isem_a)
    buf_b = (src_b, ea_b, rows_b, gsem_b, isem_b)

    def _pair(m, carry):
        k = m * 2

        @pl.when(k < nch)
        def _():
            sub_body(k, buf_a, buf_b)

        @pl.when(k + 1 < nch)
        def _():
            sub_body(k + 1, buf_b, buf_a)

        return carry

    lax.fori_loop(0, PAIRS, _pair, 0)
    wait_scatter(rows_a)
    wait_scatter(rows_b)
    plsc.subcore_barrier()

    @pl.when(cid == 0)
    def _():
        for j in range(ROWS_PER_TILE // C):
            r0 = sid * ROWS_PER_TILE + j * C
            pltpu.sync_copy(shared.at[pl.ds(r0, C)], out0_hbm.at[pl.ds(r0, C)])

    @pl.when(cid == 1)
    def _():
        for j in range(ROWS_PER_TILE // C):
            r0 = sid * ROWS_PER_TILE + j * C
            pltpu.sync_copy(shared.at[pl.ds(r0, C)], out1_hbm.at[pl.ds(r0, C)])


def _sc_aggregate(x, ei_flat, ea_t, wb):
    mesh = plsc.VectorSubcoreMesh(core_axis_name="c", subcore_axis_name="s")
    return pl.kernel(
        _sc_body,
        out_type=(jax.ShapeDtypeStruct((NPAD, D), jnp.float32),
                  jax.ShapeDtypeStruct((NPAD, D), jnp.float32)),
        mesh=mesh,
        scratch_types=[
            pltpu.VMEM((1, C), jnp.int32),
            pltpu.VMEM((DE, C), jnp.float32),
            pltpu.VMEM((C, D), jnp.float32),
            pltpu.VMEM((1, C), jnp.int32),
            pltpu.VMEM((DE, C), jnp.float32),
            pltpu.VMEM((C, D), jnp.float32),
            pltpu.VMEM((DSTN, C), jnp.int32),
            pltpu.VMEM((16,), jnp.float32),
            pltpu.VMEM_SHARED((NPAD, D), jnp.float32),
            pltpu.SemaphoreType.DMA,
            pltpu.SemaphoreType.DMA,
            pltpu.SemaphoreType.DMA,
            pltpu.SemaphoreType.DMA,
            pltpu.SemaphoreType.DMA,
        ],
    )(x, ei_flat, ea_t, wb)


def _xw_body(x_ref, wn_ref, bn_ref, o_ref):
    o_ref[...] = (jnp.dot(x_ref[...], wn_ref[...],
                          preferred_element_type=jnp.float32) + bn_ref[...])


def _xw(x, Wn, bn):
    BN = 2000
    return pl.pallas_call(
        _xw_body,
        grid=(N // BN,),
        in_specs=[
            pl.BlockSpec((BN, D), lambda i: (i, 0)),
            pl.BlockSpec((D, D), lambda i: (0, 0)),
            pl.BlockSpec((1, D), lambda i: (0, 0)),
        ],
        out_specs=pl.BlockSpec((BN, D), lambda i: (i, 0)),
        out_shape=jax.ShapeDtypeStruct((N, D), jnp.float32),
    )(x, Wn[:D], bn.reshape(1, D))


def _out_body(xw_ref, p0_ref, p1_ref, w2_ref, o_ref):
    agg = p0_ref[...] + p1_ref[...]
    o_ref[...] = xw_ref[...] + jnp.dot(agg, w2_ref[...],
                                       preferred_element_type=jnp.float32)


def _node_update(xw, p0, p1, Wn):
    BN = 2000
    return pl.pallas_call(
        _out_body,
        grid=(N // BN,),
        in_specs=[
            pl.BlockSpec((BN, D), lambda i: (i, 0)),
            pl.BlockSpec((BN, D), lambda i: (i, 0)),
            pl.BlockSpec((BN, D), lambda i: (i, 0)),
            pl.BlockSpec((D, D), lambda i: (0, 0)),
        ],
        out_specs=pl.BlockSpec((BN, D), lambda i: (i, 0)),
        out_shape=jax.ShapeDtypeStruct((N, D), jnp.float32),
    )(xw, p0, p1, Wn[D:])


def kernel(x, edge_index, edge_attr, we, beg, Wn, bn):
    ei_flat = edge_index.reshape(2 * E)
    ea_t = edge_attr.T
    wb = jnp.concatenate([we.reshape(DE), beg,
                          jnp.zeros((16 - DE - 1,), jnp.float32)])
    xw = _xw(x, Wn, bn)
    p0, p1 = _sc_aggregate(x, ei_flat, ea_t, wb)
    return _node_update(xw, p0, p1, Wn)
